# baseline (speedup 1.0000x reference)
import functools

import jax
import jax.numpy as jnp
from jax import lax
from jax.experimental import pallas as pl
from jax.experimental.pallas import tpu as pltpu
from jax.experimental.pallas import tpu_sc as plsc

SC_CORES = 2
SC_SUBCORES = 16
NW = SC_CORES * SC_SUBCORES

DIM = 768
HEADS = 12
DIM_HEAD = 64
EXPERTS = 8
HIDDEN = DIM * 4
TOP_K = 2
T = 2048
SI = 1024
BM = 128
S = TOP_K * T + EXPERTS * BM
NBLK = S // BM


def _bdot(a, b, dn=None):
    a16 = a.astype(jnp.bfloat16)
    b16 = b.astype(jnp.bfloat16)
    if dn is None:
        return jnp.dot(a16, b16, preferred_element_type=jnp.float32)
    return jax.lax.dot_general(a16, b16, dn,
                               preferred_element_type=jnp.float32)


def _ln(x, g):
    mu = jnp.mean(x, axis=-1, keepdims=True)
    xc = x - mu
    var = jnp.mean(xc * xc, axis=-1, keepdims=True)
    return xc / jnp.sqrt(var + 1e-5) * g


def _v_body(img_ref, wv_ref, v_ref):
    v_ref[...] = _bdot(img_ref[...], wv_ref[...])


def _v_proj(img, Wv):
    return pl.pallas_call(
        _v_body,
        grid=(4,),
        in_specs=[
            pl.BlockSpec((SI // 4, DIM), lambda i: (i, 0)),
            pl.BlockSpec((DIM, DIM), lambda i: (0, 0)),
        ],
        out_specs=pl.BlockSpec((SI // 4, DIM), lambda i: (i, 0)),
        out_shape=jax.ShapeDtypeStruct((SI, DIM), jnp.float32),
    )(img, Wv)


def _kt_body(wkt_ref, imgT_ref, kt_ref):
    kt_ref[...] = _bdot(wkt_ref[...], imgT_ref[...])


def _kt_proj(WkT, imgT):
    return pl.pallas_call(
        _kt_body,
        grid=(4,),
        in_specs=[
            pl.BlockSpec((DIM, DIM), lambda i: (0, 0)),
            pl.BlockSpec((DIM, SI // 4), lambda i: (0, i)),
        ],
        out_specs=pl.BlockSpec((DIM, SI // 4), lambda i: (0, i)),
        out_shape=jax.ShapeDtypeStruct((DIM, SI), jnp.float32),
    )(WkT, imgT)


def _attn_body(text_ref, kt_ref, v_ref, lnq_ref, wq_ref, nk_ref, nv_ref,
               wo_ref, lno_ref, gw_ref, act_ref, logits_ref):
    x = text_ref[...]
    xn = _ln(x, lnq_ref[...])
    scale = DIM_HEAD ** -0.5
    q = _bdot(xn, wq_ref[...]) * scale
    outs = []
    for h in range(HEADS):
        sl = slice(h * DIM_HEAD, (h + 1) * DIM_HEAD)
        q_h = q[:, sl]
        kt_h = kt_ref[sl, :]
        v_h = v_ref[:, sl]
        s = _bdot(q_h, kt_h)
        nl = _bdot(q_h, nk_ref[...])
        m = jnp.maximum(jnp.max(s, axis=1, keepdims=True), nl)
        p = jnp.exp(s - m)
        pn = jnp.exp(nl - m)
        den = jnp.sum(p, axis=1, keepdims=True) + pn
        attn = p / den
        attn_n = (pn / den).astype(jnp.bfloat16).astype(jnp.float32)
        nv16 = nv_ref[...].astype(jnp.bfloat16).astype(jnp.float32)
        o = _bdot(attn, v_h) + attn_n * nv16
        outs.append(o)
    out = jnp.concatenate(outs, axis=1)
    att = _ln(_bdot(out, wo_ref[...]), lno_ref[...])
    a = jnp.tanh(att) + x
    act_ref[...] = a
    logits_ref[...] = _bdot(a, gw_ref[...])


def _attention(text, kt, v, ln_q_g, Wq, null_k, null_v, Wo, ln_out_g, gate_W):
    BQ = 256
    return pl.pallas_call(
        _attn_body,
        grid=(T // BQ,),
        in_specs=[
            pl.BlockSpec((BQ, DIM), lambda i: (i, 0)),
            pl.BlockSpec((DIM, SI), lambda i: (0, 0)),
            pl.BlockSpec((SI, DIM), lambda i: (0, 0)),
            pl.BlockSpec((1, DIM), lambda i: (0, 0)),
            pl.BlockSpec((DIM, DIM), lambda i: (0, 0)),
            pl.BlockSpec((DIM_HEAD, 1), lambda i: (0, 0)),
            pl.BlockSpec((1, DIM_HEAD), lambda i: (0, 0)),
            pl.BlockSpec((DIM, DIM), lambda i: (0, 0)),
            pl.BlockSpec((1, DIM), lambda i: (0, 0)),
            pl.BlockSpec((DIM, EXPERTS), lambda i: (0, 0)),
        ],
        out_specs=[
            pl.BlockSpec((BQ, DIM), lambda i: (i, 0)),
            pl.BlockSpec((BQ, EXPERTS), lambda i: (i, 0)),
        ],
        out_shape=[
            jax.ShapeDtypeStruct((T, DIM), jnp.float32),
            jax.ShapeDtypeStruct((T, EXPERTS), jnp.float32),
        ],
    )(text, kt, v, ln_q_g, Wq, null_k, null_v, Wo, ln_out_g, gate_W)


def _routemath_body(logits_ref, slot_ref, w_ref, be_ref):
    x = logits_ref[...]
    m = jnp.max(x, axis=1, keepdims=True)
    eu = jnp.exp(x - m)
    z = jnp.sum(eu, axis=1, keepdims=True)
    g = eu / z
    b1 = g[:, 0:1]
    i1 = jnp.zeros_like(b1)
    b2 = jnp.full_like(b1, -1.0)
    i2 = jnp.zeros_like(b1)
    for e in range(1, EXPERTS):
        gc = g[:, e:e + 1]
        is1 = gc > b1
        is2 = jnp.logical_and(jnp.logical_not(is1), gc > b2)
        ef = jnp.full_like(b1, float(e))
        i2 = jnp.where(is1, i1, jnp.where(is2, ef, i2))
        b2 = jnp.where(is1, b1, jnp.where(is2, gc, b2))
        i1 = jnp.where(is1, ef, i1)
        b1 = jnp.where(is1, gc, b1)
    wsum = b1 + b2 + 1e-9
    w0 = b1 / wsum
    w1 = b2 / wsum
    er = jax.lax.broadcasted_iota(jnp.int32, (1, EXPERTS), 1).astype(jnp.float32)
    oh0 = (i1 == er).astype(jnp.float32)
    oh1 = (i2 == er).astype(jnp.float32)
    counts = (jnp.sum(oh0, axis=0, keepdims=True)
              + jnp.sum(oh1, axis=0, keepdims=True))
    padded = jnp.floor((counts + (BM - 1)) * (1.0 / BM)) * float(BM)
    offs = []
    run = jnp.zeros((1, 1), jnp.float32)
    for e in range(EXPERTS):
        offs.append(run)
        run = run + padded[:, e:e + 1]
    pad_off = jnp.concatenate(offs, axis=1)
    CB = 256
    r = jax.lax.broadcasted_iota(jnp.int32, (CB, CB), 0)
    c = jax.lax.broadcasted_iota(jnp.int32, (CB, CB), 1)
    tril = (r >= c).astype(jnp.float32)
    carry = jnp.zeros((1, EXPERTS), jnp.float32)
    for blk in range(2 * T // CB):
        oh = oh0 if blk < T // CB else oh1
        wv = w0 if blk < T // CB else w1
        rel = blk % (T // CB)
        ohb = oh[rel * CB:(rel + 1) * CB, :]
        incl = _bdot(tril, ohb) + carry
        rank = jnp.sum((incl - 1.0) * ohb, axis=1, keepdims=True)
        slot = jnp.sum(pad_off * ohb, axis=1, keepdims=True) + rank
        slot_ref[blk * CB:(blk + 1) * CB, :] = slot.astype(jnp.int32)
        w_ref[blk * CB:(blk + 1) * CB, :] = wv[rel * CB:(rel + 1) * CB, :]
        carry = carry + jnp.sum(ohb, axis=0, keepdims=True)
    bstart = (jax.lax.broadcasted_iota(jnp.int32, (1, 48), 1)
              .astype(jnp.float32) * float(BM))
    acc = jnp.zeros((1, 48), jnp.float32)
    for e in range(EXPERTS):
        acc = acc + (bstart >= pad_off[:, e:e + 1]).astype(jnp.float32)
    be_ref[...] = (acc - 1.0).astype(jnp.int32)


def _route_math(logits):
    return pl.pallas_call(
        _routemath_body,
        out_shape=[
            jax.ShapeDtypeStruct((2 * T, 1), jnp.int32),
            jax.ShapeDtypeStruct((2 * T, 1), jnp.float32),
            jax.ShapeDtypeStruct((1, 48), jnp.int32),
        ],
    )(logits)


def _make_scatter():
    NS = 16
    PPW = 2 * T // NS
    IPW = S // NS
    mesh = plsc.VectorSubcoreMesh(
        core_axis_name="c", subcore_axis_name="s",
        num_cores=1, num_subcores=NS)

    @functools.partial(
        pl.kernel, mesh=mesh,
        out_type=[
            jax.ShapeDtypeStruct((S,), jnp.int32),
            jax.ShapeDtypeStruct((S,), jnp.float32),
        ],
        scratch_types=[
            pltpu.VMEM((128,), jnp.int32),
            pltpu.VMEM((128,), jnp.int32),
            pltpu.VMEM((128,), jnp.int32),
            pltpu.VMEM((128,), jnp.int32),
            pltpu.VMEM((128,), jnp.float32),
            pltpu.VMEM((128,), jnp.float32),
            pltpu.VMEM((IPW,), jnp.int32),
            pltpu.VMEM((IPW,), jnp.float32),
            pltpu.SemaphoreType.DMA,
        ],
    )
    def scat(slots_hbm, w_hbm, tid_out, w_out,
             slot_a, slot_b, tid_a, tid_b, w_a, w_b, zi, zf, sem):
        wid = lax.axis_index("s")
        pbase = wid * PPW
        iota = lax.iota(jnp.int32, 16)
        zeros_i = jnp.zeros((16,), jnp.int32)
        zeros_f = jnp.zeros((16,), jnp.float32)
        for j in range(IPW // 16):
            zi[pl.ds(16 * j, 16)] = zeros_i
            zf[pl.ds(16 * j, 16)] = zeros_f
        pltpu.sync_copy(zi, tid_out.at[pl.ds(wid * IPW, IPW)])
        pltpu.sync_copy(zf, w_out.at[pl.ds(wid * IPW, IPW)])

        pltpu.sync_copy(slots_hbm.at[pl.ds(pbase, 128)], slot_a)
        pltpu.sync_copy(slots_hbm.at[pl.ds(pbase + 128, 128)], slot_b)
        pltpu.sync_copy(w_hbm.at[pl.ds(pbase, 128)], w_a)
        pltpu.sync_copy(w_hbm.at[pl.ds(pbase + 128, 128)], w_b)
        toff = lax.shift_right_arithmetic(wid, 3) * T
        for c in range(8):
            tid_a[pl.ds(16 * c, 16)] = iota + (pbase + 16 * c - toff)
            tid_b[pl.ds(16 * c, 16)] = iota + (pbase + 128 + 16 * c - toff)

        plsc.subcore_barrier()
        pltpu.async_copy(tid_a, tid_out.at[slot_a], sem).wait()
        pltpu.async_copy(tid_b, tid_out.at[slot_b], sem).wait()
        pltpu.async_copy(w_a, w_out.at[slot_a], sem).wait()
        pltpu.async_copy(w_b, w_out.at[slot_b], sem).wait()

    return scat


_scatter = _make_scatter()


def _make_row_gather(n_rows, n_tables):
    rpw = n_rows // NW
    mesh = plsc.VectorSubcoreMesh(
        core_axis_name="c", subcore_axis_name="s",
        num_cores=SC_CORES, num_subcores=SC_SUBCORES)

    @functools.partial(
        pl.kernel, mesh=mesh,
        out_type=[jax.ShapeDtypeStruct((n_rows, DIM), jnp.float32)
                  ] * n_tables,
        scratch_types=[
            pltpu.VMEM((rpw,), jnp.int32),
            pltpu.VMEM((rpw, DIM), jnp.float32),
            pltpu.SemaphoreType.DMA,
        ],
    )
    def gather(*refs):
        ins = refs[:2 * n_tables]
        outs = refs[2 * n_tables:2 * n_tables + n_tables]
        idx_v, rows_v, sem = refs[2 * n_tables + n_tables:]
        wid = lax.axis_index("s") * SC_CORES + lax.axis_index("c")
        base = wid * rpw
        for t in range(n_tables):
            table_hbm, idx_hbm = ins[2 * t], ins[2 * t + 1]
            pltpu.sync_copy(idx_hbm.at[pl.ds(base, rpw)], idx_v)
            pltpu.async_copy(table_hbm.at[idx_v], rows_v, sem).wait()
            pltpu.sync_copy(rows_v, outs[t].at[pl.ds(base, rpw)])

    return gather


_gather_sorted = _make_row_gather(S, 1)
_gather_pair = _make_row_gather(T, 2)


def _ffn_body(be_ref, x_ref, w1_ref, w2_ref, sw_ref, out_ref):
    del be_ref
    x16 = x_ref[...].astype(jnp.bfloat16)
    h = jax.nn.gelu(jnp.dot(x16, w1_ref[0],
                            preferred_element_type=jnp.float32))
    o = jnp.dot(h.astype(jnp.bfloat16), w2_ref[0],
                preferred_element_type=jnp.float32)
    out_ref[...] = o * sw_ref[0, 0][:, None]


def _ffn(block_expert, x_sorted, expert_W1, expert_W2, sorted_w3):
    grid_spec = pltpu.PrefetchScalarGridSpec(
        num_scalar_prefetch=1,
        grid=(NBLK,),
        in_specs=[
            pl.BlockSpec((BM, DIM), lambda b, be: (b, 0)),
            pl.BlockSpec((1, DIM, HIDDEN), lambda b, be: (be[b], 0, 0)),
            pl.BlockSpec((1, HIDDEN, DIM), lambda b, be: (be[b], 0, 0)),
            pl.BlockSpec((1, 1, BM), lambda b, be: (b, 0, 0)),
        ],
        out_specs=pl.BlockSpec((BM, DIM), lambda b, be: (b, 0)),
    )
    return pl.pallas_call(
        _ffn_body,
        grid_spec=grid_spec,
        out_shape=jax.ShapeDtypeStruct((S, DIM), jnp.float32),
    )(block_expert, x_sorted, expert_W1, expert_W2, sorted_w3)


def _combine_body(g0_ref, g1_ref, act_ref, out_ref):
    out_ref[...] = jnp.tanh(g0_ref[...] + g1_ref[...] + act_ref[...])


def _combine(g0, g1, activated):
    BQ = 256
    return pl.pallas_call(
        _combine_body,
        grid=(T // BQ,),
        in_specs=[pl.BlockSpec((BQ, DIM), lambda i: (i, 0))] * 3,
        out_specs=pl.BlockSpec((BQ, DIM), lambda i: (i, 0)),
        out_shape=jax.ShapeDtypeStruct((T, DIM), jnp.float32),
    )(g0, g1, activated)


def kernel(text, img, ln_q_g, Wq, Wkv, null_k, null_v, Wo, ln_out_g,
           gate_W, expert_W1, expert_W2):
    B = text.shape[0]
    text2 = text.reshape(T, DIM)
    img2 = img.reshape(SI, DIM)

    v = _v_proj(img2, Wkv[:, DIM:])
    kt = _kt_proj(Wkv[:, :DIM].T, img2.T)
    activated, logits = _attention(
        text2, kt, v, ln_q_g.reshape(1, DIM), Wq,
        null_k.reshape(DIM_HEAD, 1), null_v.reshape(1, DIM_HEAD), Wo,
        ln_out_g.reshape(1, DIM), gate_W)

    slots2, wpair2, be2 = _route_math(logits)
    slots = slots2.reshape(2 * T)
    block_expert = be2.reshape(48)[:NBLK]
    sorted_tid, sorted_w = _scatter(slots, wpair2.reshape(2 * T))
    pos0, pos1 = slots[:T], slots[T:]
    (x_sorted,) = _gather_sorted(activated, sorted_tid)
    out_sorted = _ffn(block_expert, x_sorted,
                      expert_W1.astype(jnp.bfloat16),
                      expert_W2.astype(jnp.bfloat16),
                      sorted_w.reshape(NBLK, 1, BM))
    g0, g1 = _gather_pair(out_sorted, pos0, out_sorted, pos1)
    out = _combine(g0, g1, activated)
    return out.reshape(B, T, DIM)


# speedup vs baseline: 1.2790x; 1.2790x over previous
import functools

import jax
import jax.numpy as jnp
from jax import lax
from jax.experimental import pallas as pl
from jax.experimental.pallas import tpu as pltpu
from jax.experimental.pallas import tpu_sc as plsc

SC_CORES = 2
SC_SUBCORES = 16
NW = SC_CORES * SC_SUBCORES

DIM = 768
HEADS = 12
DIM_HEAD = 64
EXPERTS = 8
HIDDEN = DIM * 4
TOP_K = 2
T = 2048
SI = 1024
BM = 128
S = TOP_K * T + EXPERTS * BM
NBLK = S // BM


def _bdot(a, b, dn=None):
    a16 = a.astype(jnp.bfloat16)
    b16 = b.astype(jnp.bfloat16)
    if dn is None:
        return jnp.dot(a16, b16, preferred_element_type=jnp.float32)
    return jax.lax.dot_general(a16, b16, dn,
                               preferred_element_type=jnp.float32)


def _ln(x, g):
    mu = jnp.mean(x, axis=-1, keepdims=True)
    xc = x - mu
    var = jnp.mean(xc * xc, axis=-1, keepdims=True)
    return xc / jnp.sqrt(var + 1e-5) * g


def _v_body(img_ref, wv_ref, v_ref):
    v_ref[...] = _bdot(img_ref[...], wv_ref[...])


def _v_proj(img, Wv):
    return pl.pallas_call(
        _v_body,
        grid=(4,),
        in_specs=[
            pl.BlockSpec((SI // 4, DIM), lambda i: (i, 0)),
            pl.BlockSpec((DIM, DIM), lambda i: (0, 0)),
        ],
        out_specs=pl.BlockSpec((SI // 4, DIM), lambda i: (i, 0)),
        out_shape=jax.ShapeDtypeStruct((SI, DIM), jnp.float32),
    )(img, Wv)


def _kt_body(wkt_ref, imgT_ref, kt_ref):
    kt_ref[...] = _bdot(wkt_ref[...], imgT_ref[...])


def _kt_proj(WkT, imgT):
    return pl.pallas_call(
        _kt_body,
        grid=(4,),
        in_specs=[
            pl.BlockSpec((DIM, DIM), lambda i: (0, 0)),
            pl.BlockSpec((DIM, SI // 4), lambda i: (0, i)),
        ],
        out_specs=pl.BlockSpec((DIM, SI // 4), lambda i: (0, i)),
        out_shape=jax.ShapeDtypeStruct((DIM, SI), jnp.float32),
    )(WkT, imgT)


def _attn_body(text_ref, kt_ref, v_ref, lnq_ref, wq_ref, nk_ref, nv_ref,
               wo_ref, lno_ref, gw_ref, act_ref, logits_ref):
    x = text_ref[...]
    xn = _ln(x, lnq_ref[...])
    scale = DIM_HEAD ** -0.5
    q = _bdot(xn, wq_ref[...]) * scale
    outs = []
    for h in range(HEADS):
        sl = slice(h * DIM_HEAD, (h + 1) * DIM_HEAD)
        q_h = q[:, sl]
        kt_h = kt_ref[sl, :]
        v_h = v_ref[:, sl]
        s = _bdot(q_h, kt_h)
        nl = _bdot(q_h, nk_ref[...])
        m = jnp.maximum(jnp.max(s, axis=1, keepdims=True), nl)
        p = jnp.exp(s - m)
        pn = jnp.exp(nl - m)
        den = jnp.sum(p, axis=1, keepdims=True) + pn
        attn = p / den
        attn_n = (pn / den).astype(jnp.bfloat16).astype(jnp.float32)
        nv16 = nv_ref[...].astype(jnp.bfloat16).astype(jnp.float32)
        o = _bdot(attn, v_h) + attn_n * nv16
        outs.append(o)
    out = jnp.concatenate(outs, axis=1)
    att = _ln(_bdot(out, wo_ref[...]), lno_ref[...])
    a = jnp.tanh(att) + x
    act_ref[...] = a
    logits_ref[...] = _bdot(a, gw_ref[...])


def _attention(text, kt, v, ln_q_g, Wq, null_k, null_v, Wo, ln_out_g, gate_W):
    BQ = 256
    return pl.pallas_call(
        _attn_body,
        grid=(T // BQ,),
        in_specs=[
            pl.BlockSpec((BQ, DIM), lambda i: (i, 0)),
            pl.BlockSpec((DIM, SI), lambda i: (0, 0)),
            pl.BlockSpec((SI, DIM), lambda i: (0, 0)),
            pl.BlockSpec((1, DIM), lambda i: (0, 0)),
            pl.BlockSpec((DIM, DIM), lambda i: (0, 0)),
            pl.BlockSpec((DIM_HEAD, 1), lambda i: (0, 0)),
            pl.BlockSpec((1, DIM_HEAD), lambda i: (0, 0)),
            pl.BlockSpec((DIM, DIM), lambda i: (0, 0)),
            pl.BlockSpec((1, DIM), lambda i: (0, 0)),
            pl.BlockSpec((DIM, EXPERTS), lambda i: (0, 0)),
        ],
        out_specs=[
            pl.BlockSpec((BQ, DIM), lambda i: (i, 0)),
            pl.BlockSpec((BQ, EXPERTS), lambda i: (i, 0)),
        ],
        out_shape=[
            jax.ShapeDtypeStruct((T, DIM), jnp.float32),
            jax.ShapeDtypeStruct((T, EXPERTS), jnp.float32),
        ],
    )(text, kt, v, ln_q_g, Wq, null_k, null_v, Wo, ln_out_g, gate_W)


def _routemath_body(logits_ref, slot_ref, w_ref, be_ref):
    x = logits_ref[...]
    m = jnp.max(x, axis=1, keepdims=True)
    eu = jnp.exp(x - m)
    z = jnp.sum(eu, axis=1, keepdims=True)
    g = eu / z
    b1 = g[:, 0:1]
    i1 = jnp.zeros_like(b1)
    b2 = jnp.full_like(b1, -1.0)
    i2 = jnp.zeros_like(b1)
    for e in range(1, EXPERTS):
        gc = g[:, e:e + 1]
        is1 = gc > b1
        is2 = jnp.logical_and(jnp.logical_not(is1), gc > b2)
        ef = jnp.full_like(b1, float(e))
        i2 = jnp.where(is1, i1, jnp.where(is2, ef, i2))
        b2 = jnp.where(is1, b1, jnp.where(is2, gc, b2))
        i1 = jnp.where(is1, ef, i1)
        b1 = jnp.where(is1, gc, b1)
    wsum = b1 + b2 + 1e-9
    w0 = b1 / wsum
    w1 = b2 / wsum
    er = jax.lax.broadcasted_iota(jnp.int32, (1, EXPERTS), 1).astype(jnp.float32)
    oh0 = (i1 == er).astype(jnp.float32)
    oh1 = (i2 == er).astype(jnp.float32)
    counts = (jnp.sum(oh0, axis=0, keepdims=True)
              + jnp.sum(oh1, axis=0, keepdims=True))
    padded = jnp.floor((counts + (BM - 1)) * (1.0 / BM)) * float(BM)
    offs = []
    run = jnp.zeros((1, 1), jnp.float32)
    for e in range(EXPERTS):
        offs.append(run)
        run = run + padded[:, e:e + 1]
    pad_off = jnp.concatenate(offs, axis=1)
    CB = 256
    r = jax.lax.broadcasted_iota(jnp.int32, (CB, CB), 0)
    c = jax.lax.broadcasted_iota(jnp.int32, (CB, CB), 1)
    tril = (r >= c).astype(jnp.float32)
    carry = jnp.zeros((1, EXPERTS), jnp.float32)
    for blk in range(2 * T // CB):
        oh = oh0 if blk < T // CB else oh1
        wv = w0 if blk < T // CB else w1
        rel = blk % (T // CB)
        ohb = oh[rel * CB:(rel + 1) * CB, :]
        incl = _bdot(tril, ohb) + carry
        rank = jnp.sum((incl - 1.0) * ohb, axis=1, keepdims=True)
        slot = jnp.sum(pad_off * ohb, axis=1, keepdims=True) + rank
        slot_ref[blk * CB:(blk + 1) * CB, :] = slot.astype(jnp.int32)
        w_ref[blk * CB:(blk + 1) * CB, :] = wv[rel * CB:(rel + 1) * CB, :]
        carry = carry + jnp.sum(ohb, axis=0, keepdims=True)
    bstart = (jax.lax.broadcasted_iota(jnp.int32, (1, 48), 1)
              .astype(jnp.float32) * float(BM))
    acc = jnp.zeros((1, 48), jnp.float32)
    for e in range(EXPERTS):
        acc = acc + (bstart >= pad_off[:, e:e + 1]).astype(jnp.float32)
    be_ref[...] = (acc - 1.0).astype(jnp.int32)


def _route_math(logits):
    return pl.pallas_call(
        _routemath_body,
        out_shape=[
            jax.ShapeDtypeStruct((2 * T, 1), jnp.int32),
            jax.ShapeDtypeStruct((2 * T, 1), jnp.float32),
            jax.ShapeDtypeStruct((1, 48), jnp.int32),
        ],
    )(logits)


def _make_scatter_rows():
    PPW = 2 * T // NW
    mesh = plsc.VectorSubcoreMesh(
        core_axis_name="c", subcore_axis_name="s",
        num_cores=SC_CORES, num_subcores=SC_SUBCORES)

    @functools.partial(
        pl.kernel, mesh=mesh,
        out_type=jax.ShapeDtypeStruct((S, DIM), jnp.float32),
        scratch_types=[
            pltpu.VMEM((PPW,), jnp.int32),
            pltpu.VMEM((PPW, DIM), jnp.float32),
            pltpu.SemaphoreType.DMA,
        ],
    )
    def scat(act_hbm, slots_hbm, xs_hbm, slotv, rows, sem):
        wid = lax.axis_index("s") * SC_CORES + lax.axis_index("c")
        pbase = wid * PPW
        toff = lax.shift_right_arithmetic(wid, 4) * T
        pltpu.sync_copy(act_hbm.at[pl.ds(pbase - toff, PPW)], rows)
        pltpu.sync_copy(slots_hbm.at[pl.ds(pbase, PPW)], slotv)
        pltpu.async_copy(rows, xs_hbm.at[slotv], sem).wait()

    return scat


_scatter_rows = _make_scatter_rows()


def _make_row_gather(n_rows, n_tables):
    rpw = n_rows // NW
    mesh = plsc.VectorSubcoreMesh(
        core_axis_name="c", subcore_axis_name="s",
        num_cores=SC_CORES, num_subcores=SC_SUBCORES)

    @functools.partial(
        pl.kernel, mesh=mesh,
        out_type=[jax.ShapeDtypeStruct((n_rows, DIM), jnp.float32)
                  ] * n_tables,
        scratch_types=[
            pltpu.VMEM((rpw,), jnp.int32),
            pltpu.VMEM((rpw, DIM), jnp.float32),
            pltpu.SemaphoreType.DMA,
        ],
    )
    def gather(*refs):
        ins = refs[:2 * n_tables]
        outs = refs[2 * n_tables:2 * n_tables + n_tables]
        idx_v, rows_v, sem = refs[2 * n_tables + n_tables:]
        wid = lax.axis_index("s") * SC_CORES + lax.axis_index("c")
        base = wid * rpw
        for t in range(n_tables):
            table_hbm, idx_hbm = ins[2 * t], ins[2 * t + 1]
            pltpu.sync_copy(idx_hbm.at[pl.ds(base, rpw)], idx_v)
            pltpu.async_copy(table_hbm.at[idx_v], rows_v, sem).wait()
            pltpu.sync_copy(rows_v, outs[t].at[pl.ds(base, rpw)])

    return gather


_gather_pair = _make_row_gather(T, 2)


def _ffn_body(be_ref, x_ref, w1_ref, w2_ref, out_ref):
    del be_ref
    x16 = x_ref[...].astype(jnp.bfloat16)
    h = jax.nn.gelu(jnp.dot(x16, w1_ref[0],
                            preferred_element_type=jnp.float32))
    o = jnp.dot(h.astype(jnp.bfloat16), w2_ref[0],
                preferred_element_type=jnp.float32)
    out_ref[...] = o


def _ffn(block_expert, x_sorted, expert_W1, expert_W2):
    grid_spec = pltpu.PrefetchScalarGridSpec(
        num_scalar_prefetch=1,
        grid=(NBLK,),
        in_specs=[
            pl.BlockSpec((BM, DIM), lambda b, be: (b, 0)),
            pl.BlockSpec((1, DIM, HIDDEN), lambda b, be: (be[b], 0, 0)),
            pl.BlockSpec((1, HIDDEN, DIM), lambda b, be: (be[b], 0, 0)),
        ],
        out_specs=pl.BlockSpec((BM, DIM), lambda b, be: (b, 0)),
    )
    return pl.pallas_call(
        _ffn_body,
        grid_spec=grid_spec,
        out_shape=jax.ShapeDtypeStruct((S, DIM), jnp.float32),
    )(block_expert, x_sorted, expert_W1, expert_W2)


def _combine_body(g0_ref, g1_ref, w0_ref, w1_ref, act_ref, out_ref):
    out_ref[...] = jnp.tanh(w0_ref[...] * g0_ref[...]
                            + w1_ref[...] * g1_ref[...] + act_ref[...])


def _combine(g0, g1, w0, w1, activated):
    BQ = 256
    return pl.pallas_call(
        _combine_body,
        grid=(T // BQ,),
        in_specs=[
            pl.BlockSpec((BQ, DIM), lambda i: (i, 0)),
            pl.BlockSpec((BQ, DIM), lambda i: (i, 0)),
            pl.BlockSpec((BQ, 1), lambda i: (i, 0)),
            pl.BlockSpec((BQ, 1), lambda i: (i, 0)),
            pl.BlockSpec((BQ, DIM), lambda i: (i, 0)),
        ],
        out_specs=pl.BlockSpec((BQ, DIM), lambda i: (i, 0)),
        out_shape=jax.ShapeDtypeStruct((T, DIM), jnp.float32),
    )(g0, g1, w0, w1, activated)


def kernel(text, img, ln_q_g, Wq, Wkv, null_k, null_v, Wo, ln_out_g,
           gate_W, expert_W1, expert_W2):
    B = text.shape[0]
    text2 = text.reshape(T, DIM)
    img2 = img.reshape(SI, DIM)

    v = _v_proj(img2, Wkv[:, DIM:])
    kt = _kt_proj(Wkv[:, :DIM].T, img2.T)
    activated, logits = _attention(
        text2, kt, v, ln_q_g.reshape(1, DIM), Wq,
        null_k.reshape(DIM_HEAD, 1), null_v.reshape(1, DIM_HEAD), Wo,
        ln_out_g.reshape(1, DIM), gate_W)

    slots2, wpair2, be2 = _route_math(logits)
    slots = slots2.reshape(2 * T)
    block_expert = be2.reshape(48)[:NBLK]
    x_sorted = _scatter_rows(activated, slots)
    pos0, pos1 = slots[:T], slots[T:]
    out_sorted = _ffn(block_expert, x_sorted,
                      expert_W1.astype(jnp.bfloat16),
                      expert_W2.astype(jnp.bfloat16))
    g0, g1 = _gather_pair(out_sorted, pos0, out_sorted, pos1)
    out = _combine(g0, g1, wpair2[:T], wpair2[T:], activated)
    return out.reshape(B, T, DIM)


# speedup vs baseline: 1.3058x; 1.0209x over previous
import functools

import jax
import jax.numpy as jnp
from jax import lax
from jax.experimental import pallas as pl
from jax.experimental.pallas import tpu as pltpu
from jax.experimental.pallas import tpu_sc as plsc

SC_CORES = 2
SC_SUBCORES = 16
NW = SC_CORES * SC_SUBCORES

DIM = 768
HEADS = 12
DIM_HEAD = 64
EXPERTS = 8
HIDDEN = DIM * 4
TOP_K = 2
T = 2048
SI = 1024
BM = 256
S = TOP_K * T + EXPERTS * BM
NBLK = S // BM


def _bdot(a, b, dn=None):
    a16 = a.astype(jnp.bfloat16)
    b16 = b.astype(jnp.bfloat16)
    if dn is None:
        return jnp.dot(a16, b16, preferred_element_type=jnp.float32)
    return jax.lax.dot_general(a16, b16, dn,
                               preferred_element_type=jnp.float32)


def _ln(x, g):
    mu = jnp.mean(x, axis=-1, keepdims=True)
    xc = x - mu
    var = jnp.mean(xc * xc, axis=-1, keepdims=True)
    return xc / jnp.sqrt(var + 1e-5) * g


def _v_body(img_ref, wv_ref, v_ref):
    v_ref[...] = _bdot(img_ref[...], wv_ref[...])


def _v_proj(img, Wv):
    return pl.pallas_call(
        _v_body,
        grid=(4,),
        in_specs=[
            pl.BlockSpec((SI // 4, DIM), lambda i: (i, 0)),
            pl.BlockSpec((DIM, DIM), lambda i: (0, 0)),
        ],
        out_specs=pl.BlockSpec((SI // 4, DIM), lambda i: (i, 0)),
        out_shape=jax.ShapeDtypeStruct((SI, DIM), jnp.float32),
    )(img, Wv)


def _kt_body(wkt_ref, imgT_ref, kt_ref):
    kt_ref[...] = _bdot(wkt_ref[...], imgT_ref[...])


def _kt_proj(WkT, imgT):
    return pl.pallas_call(
        _kt_body,
        grid=(4,),
        in_specs=[
            pl.BlockSpec((DIM, DIM), lambda i: (0, 0)),
            pl.BlockSpec((DIM, SI // 4), lambda i: (0, i)),
        ],
        out_specs=pl.BlockSpec((DIM, SI // 4), lambda i: (0, i)),
        out_shape=jax.ShapeDtypeStruct((DIM, SI), jnp.float32),
    )(WkT, imgT)


def _attn_body(text_ref, kt_ref, v_ref, lnq_ref, wq_ref, nk_ref, nv_ref,
               wo_ref, lno_ref, gw_ref, act_ref, logits_ref):
    x = text_ref[...]
    xn = _ln(x, lnq_ref[...])
    scale = DIM_HEAD ** -0.5
    q = _bdot(xn, wq_ref[...]) * scale
    outs = []
    for h in range(HEADS):
        sl = slice(h * DIM_HEAD, (h + 1) * DIM_HEAD)
        q_h = q[:, sl]
        kt_h = kt_ref[sl, :]
        v_h = v_ref[:, sl]
        s = _bdot(q_h, kt_h)
        nl = _bdot(q_h, nk_ref[...])
        m = jnp.maximum(jnp.max(s, axis=1, keepdims=True), nl)
        p = jnp.exp(s - m)
        pn = jnp.exp(nl - m)
        den = jnp.sum(p, axis=1, keepdims=True) + pn
        attn = p / den
        attn_n = (pn / den).astype(jnp.bfloat16).astype(jnp.float32)
        nv16 = nv_ref[...].astype(jnp.bfloat16).astype(jnp.float32)
        o = _bdot(attn, v_h) + attn_n * nv16
        outs.append(o)
    out = jnp.concatenate(outs, axis=1)
    att = _ln(_bdot(out, wo_ref[...]), lno_ref[...])
    a = jnp.tanh(att) + x
    act_ref[...] = a
    logits_ref[...] = _bdot(a, gw_ref[...])


def _attention(text, kt, v, ln_q_g, Wq, null_k, null_v, Wo, ln_out_g, gate_W):
    BQ = 256
    return pl.pallas_call(
        _attn_body,
        grid=(T // BQ,),
        in_specs=[
            pl.BlockSpec((BQ, DIM), lambda i: (i, 0)),
            pl.BlockSpec((DIM, SI), lambda i: (0, 0)),
            pl.BlockSpec((SI, DIM), lambda i: (0, 0)),
            pl.BlockSpec((1, DIM), lambda i: (0, 0)),
            pl.BlockSpec((DIM, DIM), lambda i: (0, 0)),
            pl.BlockSpec((DIM_HEAD, 1), lambda i: (0, 0)),
            pl.BlockSpec((1, DIM_HEAD), lambda i: (0, 0)),
            pl.BlockSpec((DIM, DIM), lambda i: (0, 0)),
            pl.BlockSpec((1, DIM), lambda i: (0, 0)),
            pl.BlockSpec((DIM, EXPERTS), lambda i: (0, 0)),
        ],
        out_specs=[
            pl.BlockSpec((BQ, DIM), lambda i: (i, 0)),
            pl.BlockSpec((BQ, EXPERTS), lambda i: (i, 0)),
        ],
        out_shape=[
            jax.ShapeDtypeStruct((T, DIM), jnp.float32),
            jax.ShapeDtypeStruct((T, EXPERTS), jnp.float32),
        ],
    )(text, kt, v, ln_q_g, Wq, null_k, null_v, Wo, ln_out_g, gate_W)


def _routemath_body(logits_ref, slot_ref, w_ref, be_ref):
    x = logits_ref[...]
    m = jnp.max(x, axis=1, keepdims=True)
    eu = jnp.exp(x - m)
    z = jnp.sum(eu, axis=1, keepdims=True)
    g = eu / z
    b1 = g[:, 0:1]
    i1 = jnp.zeros_like(b1)
    b2 = jnp.full_like(b1, -1.0)
    i2 = jnp.zeros_like(b1)
    for e in range(1, EXPERTS):
        gc = g[:, e:e + 1]
        is1 = gc > b1
        is2 = jnp.logical_and(jnp.logical_not(is1), gc > b2)
        ef = jnp.full_like(b1, float(e))
        i2 = jnp.where(is1, i1, jnp.where(is2, ef, i2))
        b2 = jnp.where(is1, b1, jnp.where(is2, gc, b2))
        i1 = jnp.where(is1, ef, i1)
        b1 = jnp.where(is1, gc, b1)
    wsum = b1 + b2 + 1e-9
    w0 = b1 / wsum
    w1 = b2 / wsum
    er = jax.lax.broadcasted_iota(jnp.int32, (1, EXPERTS), 1).astype(jnp.float32)
    oh0 = (i1 == er).astype(jnp.float32)
    oh1 = (i2 == er).astype(jnp.float32)
    counts = (jnp.sum(oh0, axis=0, keepdims=True)
              + jnp.sum(oh1, axis=0, keepdims=True))
    padded = jnp.floor((counts + (BM - 1)) * (1.0 / BM)) * float(BM)
    offs = []
    run = jnp.zeros((1, 1), jnp.float32)
    for e in range(EXPERTS):
        offs.append(run)
        run = run + padded[:, e:e + 1]
    pad_off = jnp.concatenate(offs, axis=1)
    CB = 256
    r = jax.lax.broadcasted_iota(jnp.int32, (CB, CB), 0)
    c = jax.lax.broadcasted_iota(jnp.int32, (CB, CB), 1)
    tril = (r >= c).astype(jnp.float32)
    carry = jnp.zeros((1, EXPERTS), jnp.float32)
    for blk in range(2 * T // CB):
        oh = oh0 if blk < T // CB else oh1
        wv = w0 if blk < T // CB else w1
        rel = blk % (T // CB)
        ohb = oh[rel * CB:(rel + 1) * CB, :]
        incl = _bdot(tril, ohb) + carry
        rank = jnp.sum((incl - 1.0) * ohb, axis=1, keepdims=True)
        slot = jnp.sum(pad_off * ohb, axis=1, keepdims=True) + rank
        slot_ref[blk * CB:(blk + 1) * CB, :] = slot.astype(jnp.int32)
        w_ref[blk * CB:(blk + 1) * CB, :] = wv[rel * CB:(rel + 1) * CB, :]
        carry = carry + jnp.sum(ohb, axis=0, keepdims=True)
    bstart = (jax.lax.broadcasted_iota(jnp.int32, (1, 48), 1)
              .astype(jnp.float32) * float(BM))
    acc = jnp.zeros((1, 48), jnp.float32)
    for e in range(EXPERTS):
        acc = acc + (bstart >= pad_off[:, e:e + 1]).astype(jnp.float32)
    be_ref[...] = (acc - 1.0).astype(jnp.int32)


def _route_math(logits):
    return pl.pallas_call(
        _routemath_body,
        out_shape=[
            jax.ShapeDtypeStruct((2 * T, 1), jnp.int32),
            jax.ShapeDtypeStruct((2 * T, 1), jnp.float32),
            jax.ShapeDtypeStruct((1, 48), jnp.int32),
        ],
    )(logits)


def _make_scatter_rows():
    PPW = 2 * T // NW
    mesh = plsc.VectorSubcoreMesh(
        core_axis_name="c", subcore_axis_name="s",
        num_cores=SC_CORES, num_subcores=SC_SUBCORES)

    @functools.partial(
        pl.kernel, mesh=mesh,
        out_type=jax.ShapeDtypeStruct((S, DIM), jnp.float32),
        scratch_types=[
            pltpu.VMEM((PPW,), jnp.int32),
            pltpu.VMEM((PPW, DIM), jnp.float32),
            pltpu.SemaphoreType.DMA,
        ],
    )
    def scat(act_hbm, slots_hbm, xs_hbm, slotv, rows, sem):
        wid = lax.axis_index("s") * SC_CORES + lax.axis_index("c")
        pbase = wid * PPW
        toff = lax.shift_right_arithmetic(wid, 4) * T
        pltpu.sync_copy(act_hbm.at[pl.ds(pbase - toff, PPW)], rows)
        pltpu.sync_copy(slots_hbm.at[pl.ds(pbase, PPW)], slotv)
        pltpu.async_copy(rows, xs_hbm.at[slotv], sem).wait()

    return scat


_scatter_rows = _make_scatter_rows()


def _make_row_gather(n_rows, n_tables):
    rpw = n_rows // NW
    mesh = plsc.VectorSubcoreMesh(
        core_axis_name="c", subcore_axis_name="s",
        num_cores=SC_CORES, num_subcores=SC_SUBCORES)

    @functools.partial(
        pl.kernel, mesh=mesh,
        out_type=[jax.ShapeDtypeStruct((n_rows, DIM), jnp.float32)
                  ] * n_tables,
        scratch_types=[
            pltpu.VMEM((rpw,), jnp.int32),
            pltpu.VMEM((rpw, DIM), jnp.float32),
            pltpu.SemaphoreType.DMA,
        ],
    )
    def gather(*refs):
        ins = refs[:2 * n_tables]
        outs = refs[2 * n_tables:2 * n_tables + n_tables]
        idx_v, rows_v, sem = refs[2 * n_tables + n_tables:]
        wid = lax.axis_index("s") * SC_CORES + lax.axis_index("c")
        base = wid * rpw
        for t in range(n_tables):
            table_hbm, idx_hbm = ins[2 * t], ins[2 * t + 1]
            pltpu.sync_copy(idx_hbm.at[pl.ds(base, rpw)], idx_v)
            pltpu.async_copy(table_hbm.at[idx_v], rows_v, sem).wait()
            pltpu.sync_copy(rows_v, outs[t].at[pl.ds(base, rpw)])

    return gather


_gather_pair = _make_row_gather(T, 2)


def _ffn_body(be_ref, x_ref, w1_ref, w2_ref, out_ref):
    del be_ref
    x16 = x_ref[...].astype(jnp.bfloat16)
    h = jax.nn.gelu(jnp.dot(x16, w1_ref[0],
                            preferred_element_type=jnp.float32))
    o = jnp.dot(h.astype(jnp.bfloat16), w2_ref[0],
                preferred_element_type=jnp.float32)
    out_ref[...] = o


def _ffn(block_expert, x_sorted, expert_W1, expert_W2):
    grid_spec = pltpu.PrefetchScalarGridSpec(
        num_scalar_prefetch=1,
        grid=(NBLK,),
        in_specs=[
            pl.BlockSpec((BM, DIM), lambda b, be: (b, 0)),
            pl.BlockSpec((1, DIM, HIDDEN), lambda b, be: (be[b], 0, 0)),
            pl.BlockSpec((1, HIDDEN, DIM), lambda b, be: (be[b], 0, 0)),
        ],
        out_specs=pl.BlockSpec((BM, DIM), lambda b, be: (b, 0)),
    )
    return pl.pallas_call(
        _ffn_body,
        grid_spec=grid_spec,
        out_shape=jax.ShapeDtypeStruct((S, DIM), jnp.float32),
    )(block_expert, x_sorted, expert_W1, expert_W2)


def _combine_body(g0_ref, g1_ref, w0_ref, w1_ref, act_ref, out_ref):
    out_ref[...] = jnp.tanh(w0_ref[...] * g0_ref[...]
                            + w1_ref[...] * g1_ref[...] + act_ref[...])


def _combine(g0, g1, w0, w1, activated):
    BQ = 256
    return pl.pallas_call(
        _combine_body,
        grid=(T // BQ,),
        in_specs=[
            pl.BlockSpec((BQ, DIM), lambda i: (i, 0)),
            pl.BlockSpec((BQ, DIM), lambda i: (i, 0)),
            pl.BlockSpec((BQ, 1), lambda i: (i, 0)),
            pl.BlockSpec((BQ, 1), lambda i: (i, 0)),
            pl.BlockSpec((BQ, DIM), lambda i: (i, 0)),
        ],
        out_specs=pl.BlockSpec((BQ, DIM), lambda i: (i, 0)),
        out_shape=jax.ShapeDtypeStruct((T, DIM), jnp.float32),
    )(g0, g1, w0, w1, activated)


def kernel(text, img, ln_q_g, Wq, Wkv, null_k, null_v, Wo, ln_out_g,
           gate_W, expert_W1, expert_W2):
    B = text.shape[0]
    text2 = text.reshape(T, DIM)
    img2 = img.reshape(SI, DIM)

    v = _v_proj(img2, Wkv[:, DIM:])
    kt = _kt_proj(Wkv[:, :DIM].T, img2.T)
    activated, logits = _attention(
        text2, kt, v, ln_q_g.reshape(1, DIM), Wq,
        null_k.reshape(DIM_HEAD, 1), null_v.reshape(1, DIM_HEAD), Wo,
        ln_out_g.reshape(1, DIM), gate_W)

    slots2, wpair2, be2 = _route_math(logits)
    slots = slots2.reshape(2 * T)
    block_expert = be2.reshape(48)[:NBLK]
    x_sorted = _scatter_rows(activated, slots)
    pos0, pos1 = slots[:T], slots[T:]
    out_sorted = _ffn(block_expert, x_sorted,
                      expert_W1.astype(jnp.bfloat16),
                      expert_W2.astype(jnp.bfloat16))
    g0, g1 = _gather_pair(out_sorted, pos0, out_sorted, pos1)
    out = _combine(g0, g1, wpair2[:T], wpair2[T:], activated)
    return out.reshape(B, T, DIM)
